# Initial kernel scaffold; baseline (speedup 1.0000x reference)
import jax
import jax.numpy as jnp
from jax.experimental import pallas as pl


def kernel(x, t, mask, Wa, Wb, Wc, Wd, Wp):
    raise NotImplementedError("write your pallas kernel here")


# speedup vs baseline: 6.8245x; 6.8245x over previous
import functools

import numpy as np
import jax
import jax.numpy as jnp
from jax import lax
from jax.experimental import pallas as pl
from jax.experimental.pallas import tpu as pltpu
from jax.experimental.pallas import tpu_sc as plsc

_NUM_BANDS = 2
_EMB = 64
_NH = 4
_TMAX = 1000.0
_B, _L = 16, 2048
_L2 = _L * _NUM_BANDS
_CH = 1024
_HARMONICS = np.arange(1, _NH + 1, dtype=np.float32) * np.float32(
    2.0 * np.pi / _TMAX)


def _film_body(x_ref, t_ref, wa_ref, wb_ref, wc_ref, wd_ref, wp_ref, o_ref):
    band = pl.program_id(1)
    is0 = band == 0
    xb = jnp.where(is0, x_ref[0, :, 0:1], x_ref[0, :, 1:2])
    tb = jnp.where(is0, t_ref[0, :, 0:1], t_ref[0, :, 1:2])
    wa = jnp.where(is0, wa_ref[0], wa_ref[1])
    wb = jnp.where(is0, wb_ref[0], wb_ref[1])
    wc = jnp.where(is0, wc_ref[0], wc_ref[1])
    wd = jnp.where(is0, wd_ref[0], wd_ref[1])
    wp = jnp.where(is0, wp_ref[0], wp_ref[1])
    arg = jnp.concatenate([tb * float(h) for h in _HARMONICS], axis=1)
    s = jnp.sin(arg)
    c = jnp.cos(arg)
    alpha = (jnp.dot(s, wa, preferred_element_type=jnp.float32)
             + jnp.dot(c, wb, preferred_element_type=jnp.float32))
    beta = (jnp.dot(s, wc, preferred_element_type=jnp.float32)
            + jnp.dot(c, wd, preferred_element_type=jnp.float32))
    o_ref[0] = alpha * (xb * wp) + beta


def _film_tc(x, t, Wa, Wb, Wc, Wd, Wp):
    full_w = pl.BlockSpec((_NUM_BANDS, _NH, _EMB), lambda b, k: (0, 0, 0))
    full_p = pl.BlockSpec((_NUM_BANDS, 1, _EMB), lambda b, k: (0, 0, 0))
    return pl.pallas_call(
        _film_body,
        grid=(_B, _NUM_BANDS),
        in_specs=[
            pl.BlockSpec((1, _L, _NUM_BANDS), lambda b, k: (b, 0, 0)),
            pl.BlockSpec((1, _L, _NUM_BANDS), lambda b, k: (b, 0, 0)),
            full_w, full_w, full_w, full_w, full_p,
        ],
        out_specs=pl.BlockSpec((1, _L, _EMB), lambda b, k: (b, k, 0)),
        out_shape=jax.ShapeDtypeStruct((_B, _L2, _EMB), jnp.float32),
    )(x, t, Wa, Wb, Wc, Wd, Wp)


def _partition_body(xmod_hbm, m_hbm, t_hbm, xs_hbm, ms_hbm, ts_hbm,
                    inbuf, outbuf, colin, colout):
    cid = lax.axis_index("c")
    sid = lax.axis_index("s")
    wid = sid * 2 + cid
    lanes = lax.iota(jnp.int32, 16)

    for task in range(2):
        tid = wid * 2 + task
        b = tid // 4
        col0 = (tid % 4) * 16
        carry_nz = jnp.zeros((16,), jnp.int32)
        carry_z = jnp.zeros((16,), jnp.int32)
        for ci in range(_L2 // _CH):
            pltpu.sync_copy(
                xmod_hbm.at[b, pl.ds(ci * _CH, _CH), pl.ds(col0, 16)], inbuf)

            def row_body(r, carr):
                cnz, cz = carr
                v = inbuf[r]
                nz = v != 0.0
                one = jnp.where(nz, 1, 0).astype(jnp.int32)
                dst = jnp.where(nz, cnz, (_L2 - 1) - cz)
                plsc.store_scatter(outbuf, [dst, lanes], v)
                return (cnz + one, cz + (1 - one))

            carry_nz, carry_z = lax.fori_loop(
                0, _CH, row_body, (carry_nz, carry_z))
        pltpu.sync_copy(outbuf, xs_hbm.at[b, :, pl.ds(col0, 16)])

    def column_task(src_hbm, dst_hbm, row):
        pltpu.sync_copy(src_hbm.at[row], colin)

        def chunk_body(k, carr):
            cnz, cz = carr
            v = colin[pl.ds(k * 16, 16)]
            nz = v != 0.0
            one = jnp.where(nz, 1, 0).astype(jnp.int32)
            inc = plsc.cumsum(one)
            dst = jnp.where(nz, cnz + inc - 1, _L2 - 1 - cz - lanes + inc)
            plsc.store_scatter(colout, [dst], v)
            tot = jnp.sum(one)
            return (cnz + tot, cz + (16 - tot))

        lax.fori_loop(0, _L2 // 16, chunk_body,
                      (jnp.int32(0), jnp.int32(0)))
        pltpu.sync_copy(colout, dst_hbm.at[row])

    @pl.when(wid < 16)
    def _():
        column_task(m_hbm, ms_hbm, wid)

    @pl.when(wid >= 16)
    def _():
        column_task(t_hbm, ts_hbm, wid - 16)


def _partition_sc(xmod, m2, t2):
    mesh = plsc.VectorSubcoreMesh(core_axis_name="c", subcore_axis_name="s")
    f32 = jnp.float32
    run = functools.partial(
        pl.kernel,
        mesh=mesh,
        compiler_params=pltpu.CompilerParams(
            use_tc_tiling_on_sc=False, needs_layout_passes=False),
        out_type=(
            jax.ShapeDtypeStruct((_B, _L2, _EMB), f32),
            jax.ShapeDtypeStruct((_B, _L2), f32),
            jax.ShapeDtypeStruct((_B, _L2), f32),
        ),
        scratch_types=[
            pltpu.VMEM((_CH, 16), f32),
            pltpu.VMEM((_L2, 16), f32),
            pltpu.VMEM((_L2,), f32),
            pltpu.VMEM((_L2,), f32),
        ],
    )(_partition_body)
    return run(xmod, m2, t2)


def kernel(x, t, mask, Wa, Wb, Wc, Wd, Wp):
    xmod = _film_tc(x, t, Wa, Wb, Wc, Wd, Wp)
    m2 = mask.reshape(_B, _L2)
    t2 = t.reshape(_B, _L2)
    xs, ms, ts = _partition_sc(xmod, m2, t2)
    return xs, ms.reshape(_B, _L2, 1), ts.reshape(_B, _L2, 1)


# speedup vs baseline: 8.4786x; 1.2424x over previous
import functools

import numpy as np
import jax
import jax.numpy as jnp
from jax import lax
from jax.experimental import pallas as pl
from jax.experimental.pallas import tpu as pltpu
from jax.experimental.pallas import tpu_sc as plsc

_NUM_BANDS = 2
_EMB = 64
_NH = 4
_TMAX = 1000.0
_B, _L = 16, 2048
_L2 = _L * _NUM_BANDS
_CH = 1024
_HARMONICS = np.arange(1, _NH + 1, dtype=np.float32) * np.float32(
    2.0 * np.pi / _TMAX)


def _film_body(x_ref, t_ref, wa_ref, wb_ref, wc_ref, wd_ref, wp_ref, o_ref):
    band = pl.program_id(1)
    is0 = band == 0
    xb = jnp.where(is0, x_ref[0, :, 0:1], x_ref[0, :, 1:2])
    tb = jnp.where(is0, t_ref[0, :, 0:1], t_ref[0, :, 1:2])
    wa = jnp.where(is0, wa_ref[0], wa_ref[1])
    wb = jnp.where(is0, wb_ref[0], wb_ref[1])
    wc = jnp.where(is0, wc_ref[0], wc_ref[1])
    wd = jnp.where(is0, wd_ref[0], wd_ref[1])
    wp = jnp.where(is0, wp_ref[0], wp_ref[1])
    arg = jnp.concatenate([tb * float(h) for h in _HARMONICS], axis=1)
    s = jnp.sin(arg)
    c = jnp.cos(arg)
    alpha = (jnp.dot(s, wa, preferred_element_type=jnp.float32)
             + jnp.dot(c, wb, preferred_element_type=jnp.float32))
    beta = (jnp.dot(s, wc, preferred_element_type=jnp.float32)
            + jnp.dot(c, wd, preferred_element_type=jnp.float32))
    o_ref[0] = alpha * (xb * wp) + beta


def _film_tc(x, t, Wa, Wb, Wc, Wd, Wp):
    full_w = pl.BlockSpec((_NUM_BANDS, _NH, _EMB), lambda b, k: (0, 0, 0))
    full_p = pl.BlockSpec((_NUM_BANDS, 1, _EMB), lambda b, k: (0, 0, 0))
    return pl.pallas_call(
        _film_body,
        grid=(_B, _NUM_BANDS),
        in_specs=[
            pl.BlockSpec((1, _L, _NUM_BANDS), lambda b, k: (b, 0, 0)),
            pl.BlockSpec((1, _L, _NUM_BANDS), lambda b, k: (b, 0, 0)),
            full_w, full_w, full_w, full_w, full_p,
        ],
        out_specs=pl.BlockSpec((1, _L, _EMB), lambda b, k: (b, k, 0)),
        out_shape=jax.ShapeDtypeStruct((_B, _L2, _EMB), jnp.float32),
    )(x, t, Wa, Wb, Wc, Wd, Wp)


_DET_UNROLL = 8


def _has_zero(load_row, nrows):
    def det_body(r, acc):
        base = r * _DET_UNROLL
        for u in range(_DET_UNROLL):
            acc = jnp.minimum(acc, jnp.abs(load_row(base + u)))
        return acc
    acc = lax.fori_loop(0, nrows // _DET_UNROLL, det_body,
                        jnp.full((16,), 3.0e38, jnp.float32))
    return jnp.min(acc) == 0.0


def _partition_body(xmod_hbm, m_hbm, t_hbm, xs_hbm, ms_hbm, ts_hbm,
                    inbuf, outbuf, colin, colout, cnz_ref, cz_ref):
    cid = lax.axis_index("c")
    sid = lax.axis_index("s")
    wid = sid * 2 + cid
    lanes = lax.iota(jnp.int32, 16)

    for task in range(2):
        tid = wid * 2 + task
        b = tid // 4
        col0 = (tid % 4) * 16
        clean = jnp.bool_(True)
        fast_flags = []
        for ci in range(_L2 // _CH):
            pltpu.sync_copy(
                xmod_hbm.at[b, pl.ds(ci * _CH, _CH), pl.ds(col0, 16)], inbuf)
            clean_now = jnp.logical_and(
                clean, jnp.logical_not(_has_zero(lambda r: inbuf[r], _CH)))

            @pl.when(clean_now)
            def _():
                pltpu.sync_copy(
                    inbuf, xs_hbm.at[b, pl.ds(ci * _CH, _CH),
                                     pl.ds(col0, 16)])

            @pl.when(jnp.logical_not(clean_now))
            def _(clean=clean, ci=ci):
                cnz0 = jnp.where(clean,
                                 jnp.full((16,), ci * _CH, jnp.int32),
                                 cnz_ref[...])
                cz0 = jnp.where(clean, jnp.zeros((16,), jnp.int32),
                                cz_ref[...])

                def row_body(r, carr):
                    cnz, cz = carr
                    v = inbuf[r]
                    nz = v != 0.0
                    one = jnp.where(nz, 1, 0).astype(jnp.int32)
                    dst = jnp.where(nz, cnz, (_L2 - 1) - cz)
                    plsc.store_scatter(outbuf, [dst, lanes], v)
                    return (cnz + one, cz + (1 - one))

                cnz1, cz1 = lax.fori_loop(0, _CH, row_body, (cnz0, cz0))
                cnz_ref[...] = cnz1
                cz_ref[...] = cz1

            clean = clean_now
            fast_flags.append(clean_now)
        for ci in range(_L2 // _CH):
            @pl.when(jnp.logical_not(fast_flags[ci]))
            def _(ci=ci):
                pltpu.sync_copy(
                    outbuf.at[pl.ds(ci * _CH, _CH)],
                    xs_hbm.at[b, pl.ds(ci * _CH, _CH), pl.ds(col0, 16)])

    def column_task(src_hbm, dst_hbm, row):
        pltpu.sync_copy(src_hbm.at[row], colin)
        z = _has_zero(lambda r: colin[pl.ds(r * 16, 16)], _L2 // 16)

        @pl.when(jnp.logical_not(z))
        def _():
            pltpu.sync_copy(colin, dst_hbm.at[row])

        @pl.when(z)
        def _():
            def chunk_body(k, carr):
                cnz, cz = carr
                v = colin[pl.ds(k * 16, 16)]
                nz = v != 0.0
                one = jnp.where(nz, 1, 0).astype(jnp.int32)
                inc = plsc.cumsum(one)
                dst = jnp.where(nz, cnz + inc - 1,
                                _L2 - 1 - cz - lanes + inc)
                plsc.store_scatter(colout, [dst], v)
                tot = jnp.sum(one)
                return (cnz + tot, cz + (16 - tot))

            lax.fori_loop(0, _L2 // 16, chunk_body,
                          (jnp.int32(0), jnp.int32(0)))
            pltpu.sync_copy(colout, dst_hbm.at[row])

    @pl.when(wid < 16)
    def _():
        column_task(m_hbm, ms_hbm, wid)

    @pl.when(wid >= 16)
    def _():
        column_task(t_hbm, ts_hbm, wid - 16)


def _partition_sc(xmod, m2, t2):
    mesh = plsc.VectorSubcoreMesh(core_axis_name="c", subcore_axis_name="s")
    f32 = jnp.float32
    run = functools.partial(
        pl.kernel,
        mesh=mesh,
        compiler_params=pltpu.CompilerParams(
            use_tc_tiling_on_sc=False, needs_layout_passes=False),
        out_type=(
            jax.ShapeDtypeStruct((_B, _L2, _EMB), f32),
            jax.ShapeDtypeStruct((_B, _L2), f32),
            jax.ShapeDtypeStruct((_B, _L2), f32),
        ),
        scratch_types=[
            pltpu.VMEM((_CH, 16), f32),
            pltpu.VMEM((_L2, 16), f32),
            pltpu.VMEM((_L2,), f32),
            pltpu.VMEM((_L2,), f32),
            pltpu.VMEM((16,), jnp.int32),
            pltpu.VMEM((16,), jnp.int32),
        ],
    )(_partition_body)
    return run(xmod, m2, t2)


def kernel(x, t, mask, Wa, Wb, Wc, Wd, Wp):
    xmod = _film_tc(x, t, Wa, Wb, Wc, Wd, Wp)
    m2 = mask.reshape(_B, _L2)
    t2 = t.reshape(_B, _L2)
    xs, ms, ts = _partition_sc(xmod, m2, t2)
    return xs, ms.reshape(_B, _L2, 1), ts.reshape(_B, _L2, 1)


# speedup vs baseline: 16.5678x; 1.9541x over previous
import functools

import numpy as np
import jax
import jax.numpy as jnp
from jax import lax
from jax.experimental import pallas as pl
from jax.experimental.pallas import tpu as pltpu
from jax.experimental.pallas import tpu_sc as plsc

_NUM_BANDS = 2
_EMB = 64
_NH = 4
_TMAX = 1000.0
_B, _L = 16, 2048
_L2 = _L * _NUM_BANDS
_CH = 1024
_HARMONICS = np.arange(1, _NH + 1, dtype=np.float32) * np.float32(
    2.0 * np.pi / _TMAX)


def _film_body(xt_ref, tt_ref, wab_ref, wcd_ref, o_ref):
    tb = tt_ref[0, 0]
    xb = xt_ref[0, 0]
    w0 = float(_HARMONICS[0])
    s1 = jnp.sin(tb * w0)
    c1 = jnp.cos(tb * w0)
    s2 = 2.0 * s1 * c1
    c2 = 1.0 - 2.0 * s1 * s1
    s3 = s2 * c1 + c2 * s1
    c3 = c2 * c1 - s2 * s1
    s4 = 2.0 * s2 * c2
    c4 = 1.0 - 2.0 * s2 * s2
    sc = jnp.concatenate([s1, s2, s3, s4, c1, c2, c3, c4], axis=0)
    alpha_t = lax.dot_general(wab_ref[0], sc, (((1,), (0,)), ((), ())),
                              preferred_element_type=jnp.float32)
    beta_t = lax.dot_general(wcd_ref[0], sc, (((1,), (0,)), ((), ())),
                             preferred_element_type=jnp.float32)
    out_t = alpha_t * xb + beta_t
    o_ref[0] = out_t.T


def _film_tc(xT, tT, Wab, Wcd):
    return pl.pallas_call(
        _film_body,
        grid=(_B, _NUM_BANDS),
        in_specs=[
            pl.BlockSpec((1, 1, 1, _L), lambda b, k: (k, b, 0, 0)),
            pl.BlockSpec((1, 1, 1, _L), lambda b, k: (k, b, 0, 0)),
            pl.BlockSpec((1, _EMB, 2 * _NH), lambda b, k: (k, 0, 0)),
            pl.BlockSpec((1, _EMB, 2 * _NH), lambda b, k: (k, 0, 0)),
        ],
        out_specs=pl.BlockSpec((1, _L, _EMB), lambda b, k: (b, k, 0)),
        out_shape=jax.ShapeDtypeStruct((_B, _L2, _EMB), jnp.float32),
    )(xT, tT, Wab, Wcd)


_DET_UNROLL = 8


def _has_zero(load_row, nrows):
    def det_body(r, acc):
        base = r * _DET_UNROLL
        for u in range(_DET_UNROLL):
            acc = jnp.minimum(acc, jnp.abs(load_row(base + u)))
        return acc
    acc = lax.fori_loop(0, nrows // _DET_UNROLL, det_body,
                        jnp.full((16,), 3.0e38, jnp.float32))
    return jnp.min(acc) == 0.0


def _partition_body(xmod_hbm, m_hbm, t_hbm, xs_hbm, ms_hbm, ts_hbm,
                    inbuf, outbuf, colin, colout, cnz_ref, cz_ref):
    cid = lax.axis_index("c")
    sid = lax.axis_index("s")
    wid = sid * 2 + cid
    lanes = lax.iota(jnp.int32, 16)

    for task in range(2):
        tid = wid * 2 + task
        b = tid // 4
        col0 = (tid % 4) * 16
        clean = jnp.bool_(True)
        fast_flags = []
        for ci in range(_L2 // _CH):
            pltpu.sync_copy(
                xmod_hbm.at[b, pl.ds(ci * _CH, _CH), pl.ds(col0, 16)], inbuf)
            clean_now = jnp.logical_and(
                clean, jnp.logical_not(_has_zero(lambda r: inbuf[r], _CH)))

            @pl.when(clean_now)
            def _():
                pltpu.sync_copy(
                    inbuf, xs_hbm.at[b, pl.ds(ci * _CH, _CH),
                                     pl.ds(col0, 16)])

            @pl.when(jnp.logical_not(clean_now))
            def _(clean=clean, ci=ci):
                cnz0 = jnp.where(clean,
                                 jnp.full((16,), ci * _CH, jnp.int32),
                                 cnz_ref[...])
                cz0 = jnp.where(clean, jnp.zeros((16,), jnp.int32),
                                cz_ref[...])

                def row_body(r, carr):
                    cnz, cz = carr
                    v = inbuf[r]
                    nz = v != 0.0
                    one = jnp.where(nz, 1, 0).astype(jnp.int32)
                    dst = jnp.where(nz, cnz, (_L2 - 1) - cz)
                    plsc.store_scatter(outbuf, [dst, lanes], v)
                    return (cnz + one, cz + (1 - one))

                cnz1, cz1 = lax.fori_loop(0, _CH, row_body, (cnz0, cz0))
                cnz_ref[...] = cnz1
                cz_ref[...] = cz1

            clean = clean_now
            fast_flags.append(clean_now)
        for ci in range(_L2 // _CH):
            @pl.when(jnp.logical_not(fast_flags[ci]))
            def _(ci=ci):
                pltpu.sync_copy(
                    outbuf.at[pl.ds(ci * _CH, _CH)],
                    xs_hbm.at[b, pl.ds(ci * _CH, _CH), pl.ds(col0, 16)])

    def column_task(src_hbm, dst_hbm, row):
        pltpu.sync_copy(src_hbm.at[row], colin)
        z = _has_zero(lambda r: colin[pl.ds(r * 16, 16)], _L2 // 16)

        @pl.when(jnp.logical_not(z))
        def _():
            pltpu.sync_copy(colin, dst_hbm.at[row])

        @pl.when(z)
        def _():
            def chunk_body(k, carr):
                cnz, cz = carr
                v = colin[pl.ds(k * 16, 16)]
                nz = v != 0.0
                one = jnp.where(nz, 1, 0).astype(jnp.int32)
                inc = plsc.cumsum(one)
                dst = jnp.where(nz, cnz + inc - 1,
                                _L2 - 1 - cz - lanes + inc)
                plsc.store_scatter(colout, [dst], v)
                tot = jnp.sum(one)
                return (cnz + tot, cz + (16 - tot))

            lax.fori_loop(0, _L2 // 16, chunk_body,
                          (jnp.int32(0), jnp.int32(0)))
            pltpu.sync_copy(colout, dst_hbm.at[row])

    @pl.when(wid < 16)
    def _():
        column_task(m_hbm, ms_hbm, wid)

    @pl.when(wid >= 16)
    def _():
        column_task(t_hbm, ts_hbm, wid - 16)


def _partition_sc(xmod, m2, t2):
    mesh = plsc.VectorSubcoreMesh(core_axis_name="c", subcore_axis_name="s")
    f32 = jnp.float32
    run = functools.partial(
        pl.kernel,
        mesh=mesh,
        compiler_params=pltpu.CompilerParams(
            use_tc_tiling_on_sc=False, needs_layout_passes=False),
        out_type=(
            jax.ShapeDtypeStruct((_B, _L2, _EMB), f32),
            jax.ShapeDtypeStruct((_B, _L2), f32),
            jax.ShapeDtypeStruct((_B, _L2), f32),
        ),
        scratch_types=[
            pltpu.VMEM((_CH, 16), f32),
            pltpu.VMEM((_L2, 16), f32),
            pltpu.VMEM((_L2,), f32),
            pltpu.VMEM((_L2,), f32),
            pltpu.VMEM((16,), jnp.int32),
            pltpu.VMEM((16,), jnp.int32),
        ],
    )(_partition_body)
    return run(xmod, m2, t2)


def kernel(x, t, mask, Wa, Wb, Wc, Wd, Wp):
    xT = jnp.transpose(x, (2, 0, 1)).reshape(_NUM_BANDS, _B, 1, _L)
    tT = jnp.transpose(t, (2, 0, 1)).reshape(_NUM_BANDS, _B, 1, _L)
    Wab = jnp.swapaxes(jnp.concatenate([Wa * Wp, Wb * Wp], axis=1), 1, 2)
    Wcd = jnp.swapaxes(jnp.concatenate([Wc, Wd], axis=1), 1, 2)
    xmod = _film_tc(xT, tT, Wab, Wcd)
    m2 = mask.reshape(_B, _L2)
    t2 = t.reshape(_B, _L2)
    xs, ms, ts = _partition_sc(xmod, m2, t2)
    return xs, ms.reshape(_B, _L2, 1), ts.reshape(_B, _L2, 1)


# speedup vs baseline: 17.3991x; 1.0502x over previous
import functools

import numpy as np
import jax
import jax.numpy as jnp
from jax import lax
from jax.experimental import pallas as pl
from jax.experimental.pallas import tpu as pltpu
from jax.experimental.pallas import tpu_sc as plsc

_NUM_BANDS = 2
_EMB = 64
_NH = 4
_TMAX = 1000.0
_B, _L = 16, 2048
_L2 = _L * _NUM_BANDS
_CH = 1024
_HARMONICS = np.arange(1, _NH + 1, dtype=np.float32) * np.float32(
    2.0 * np.pi / _TMAX)


def _film_body(xt_ref, tt_ref, wab_ref, wcd_ref, o_ref):
    tb = tt_ref[0, 0]
    xb = xt_ref[0, 0]
    w0 = float(_HARMONICS[0])
    s1 = jnp.sin(tb * w0)
    c1 = jnp.cos(tb * w0)
    s2 = 2.0 * s1 * c1
    c2 = 1.0 - 2.0 * s1 * s1
    s3 = s2 * c1 + c2 * s1
    c3 = c2 * c1 - s2 * s1
    s4 = 2.0 * s2 * c2
    c4 = 1.0 - 2.0 * s2 * s2
    sc = jnp.concatenate([s1, s2, s3, s4, c1, c2, c3, c4], axis=0)
    alpha_t = lax.dot_general(wab_ref[0], sc, (((1,), (0,)), ((), ())),
                              preferred_element_type=jnp.float32)
    beta_t = lax.dot_general(wcd_ref[0], sc, (((1,), (0,)), ((), ())),
                             preferred_element_type=jnp.float32)
    out_t = alpha_t * xb + beta_t
    o_ref[0] = out_t.T


def _film_tc(xT, tT, Wab, Wcd):
    return pl.pallas_call(
        _film_body,
        grid=(_B, _NUM_BANDS),
        in_specs=[
            pl.BlockSpec((1, 1, 1, _L), lambda b, k: (k, b, 0, 0)),
            pl.BlockSpec((1, 1, 1, _L), lambda b, k: (k, b, 0, 0)),
            pl.BlockSpec((1, _EMB, 2 * _NH), lambda b, k: (k, 0, 0)),
            pl.BlockSpec((1, _EMB, 2 * _NH), lambda b, k: (k, 0, 0)),
        ],
        out_specs=pl.BlockSpec((1, _L, _EMB), lambda b, k: (b, k, 0)),
        out_shape=jax.ShapeDtypeStruct((_B, _L2, _EMB), jnp.float32),
    )(xT, tT, Wab, Wcd)


_DET_UNROLL = 8


def _has_zero(load_row, nrows):
    def det_body(r, acc):
        base = r * _DET_UNROLL
        for u in range(_DET_UNROLL):
            acc = jnp.minimum(acc, jnp.abs(load_row(base + u)))
        return acc
    acc = lax.fori_loop(0, nrows // _DET_UNROLL, det_body,
                        jnp.full((16,), 3.0e38, jnp.float32))
    return jnp.min(acc) == 0.0


def _partition_body(xmod_hbm, m_hbm, t_hbm, xs_hbm, ms_hbm, ts_hbm,
                    inbuf0, inbuf1, outbuf, colin, colout, cnz_ref, cz_ref,
                    sin0, sin1, sout0, sout1):
    cid = lax.axis_index("c")
    sid = lax.axis_index("s")
    wid = sid * 2 + cid
    lanes = lax.iota(jnp.int32, 16)
    bufs = (inbuf0, inbuf1)
    sins = (sin0, sin1)
    souts = (sout0, sout1)
    nch = _L2 // _CH

    for task in range(2):
        tid = wid * 2 + task
        b = tid // 4
        col0 = (tid % 4) * 16

        def src(ci):
            return xmod_hbm.at[b, pl.ds(ci * _CH, _CH), pl.ds(col0, 16)]

        def dst(ci):
            return xs_hbm.at[b, pl.ds(ci * _CH, _CH), pl.ds(col0, 16)]

        clean = jnp.bool_(True)
        fast_flags = []
        in_handles = {0: pltpu.async_copy(src(0), bufs[0], sins[0])}
        out_handles = {}
        for ci in range(nch):
            buf = bufs[ci % 2]
            if ci + 1 < nch:
                if ci >= 1:
                    @pl.when(fast_flags[ci - 1])
                    def _():
                        out_handles[ci - 1].wait()
                in_handles[ci + 1] = pltpu.async_copy(
                    src(ci + 1), bufs[(ci + 1) % 2], sins[(ci + 1) % 2])
            in_handles[ci].wait()
            clean_now = jnp.logical_and(
                clean, jnp.logical_not(_has_zero(lambda r: buf[r], _CH)))

            @pl.when(clean_now)
            def _():
                out_handles[ci] = pltpu.async_copy(
                    buf, dst(ci), souts[ci % 2])

            @pl.when(jnp.logical_not(clean_now))
            def _(clean=clean, ci=ci, buf=buf):
                cnz0 = jnp.where(clean,
                                 jnp.full((16,), ci * _CH, jnp.int32),
                                 cnz_ref[...])
                cz0 = jnp.where(clean, jnp.zeros((16,), jnp.int32),
                                cz_ref[...])

                def row_body(r, carr):
                    cnz, cz = carr
                    v = buf[r]
                    nz = v != 0.0
                    one = jnp.where(nz, 1, 0).astype(jnp.int32)
                    dst_v = jnp.where(nz, cnz, (_L2 - 1) - cz)
                    plsc.store_scatter(outbuf, [dst_v, lanes], v)
                    return (cnz + one, cz + (1 - one))

                cnz1, cz1 = lax.fori_loop(0, _CH, row_body, (cnz0, cz0))
                cnz_ref[...] = cnz1
                cz_ref[...] = cz1

            clean = clean_now
            fast_flags.append(clean_now)
        for ci in (nch - 2, nch - 1):
            @pl.when(fast_flags[ci])
            def _(ci=ci):
                out_handles[ci].wait()
        for ci in range(nch):
            @pl.when(jnp.logical_not(fast_flags[ci]))
            def _(ci=ci):
                pltpu.sync_copy(outbuf.at[pl.ds(ci * _CH, _CH)], dst(ci))

    def column_task(src_hbm, dst_hbm, row):
        pltpu.sync_copy(src_hbm.at[row], colin)
        z = _has_zero(lambda r: colin[pl.ds(r * 16, 16)], _L2 // 16)

        @pl.when(jnp.logical_not(z))
        def _():
            pltpu.sync_copy(colin, dst_hbm.at[row])

        @pl.when(z)
        def _():
            def chunk_body(k, carr):
                cnz, cz = carr
                v = colin[pl.ds(k * 16, 16)]
                nz = v != 0.0
                one = jnp.where(nz, 1, 0).astype(jnp.int32)
                inc = plsc.cumsum(one)
                dst = jnp.where(nz, cnz + inc - 1,
                                _L2 - 1 - cz - lanes + inc)
                plsc.store_scatter(colout, [dst], v)
                tot = jnp.sum(one)
                return (cnz + tot, cz + (16 - tot))

            lax.fori_loop(0, _L2 // 16, chunk_body,
                          (jnp.int32(0), jnp.int32(0)))
            pltpu.sync_copy(colout, dst_hbm.at[row])

    @pl.when(wid < 16)
    def _():
        column_task(m_hbm, ms_hbm, wid)

    @pl.when(wid >= 16)
    def _():
        column_task(t_hbm, ts_hbm, wid - 16)


def _partition_sc(xmod, m2, t2):
    mesh = plsc.VectorSubcoreMesh(core_axis_name="c", subcore_axis_name="s")
    f32 = jnp.float32
    run = functools.partial(
        pl.kernel,
        mesh=mesh,
        compiler_params=pltpu.CompilerParams(
            use_tc_tiling_on_sc=False, needs_layout_passes=False),
        out_type=(
            jax.ShapeDtypeStruct((_B, _L2, _EMB), f32),
            jax.ShapeDtypeStruct((_B, _L2), f32),
            jax.ShapeDtypeStruct((_B, _L2), f32),
        ),
        scratch_types=[
            pltpu.VMEM((_CH, 16), f32),
            pltpu.VMEM((_CH, 16), f32),
            pltpu.VMEM((_L2, 16), f32),
            pltpu.VMEM((_L2,), f32),
            pltpu.VMEM((_L2,), f32),
            pltpu.VMEM((16,), jnp.int32),
            pltpu.VMEM((16,), jnp.int32),
            pltpu.SemaphoreType.DMA,
            pltpu.SemaphoreType.DMA,
            pltpu.SemaphoreType.DMA,
            pltpu.SemaphoreType.DMA,
        ],
    )(_partition_body)
    return run(xmod, m2, t2)


def kernel(x, t, mask, Wa, Wb, Wc, Wd, Wp):
    xT = jnp.transpose(x, (2, 0, 1)).reshape(_NUM_BANDS, _B, 1, _L)
    tT = jnp.transpose(t, (2, 0, 1)).reshape(_NUM_BANDS, _B, 1, _L)
    Wab = jnp.swapaxes(jnp.concatenate([Wa * Wp, Wb * Wp], axis=1), 1, 2)
    Wcd = jnp.swapaxes(jnp.concatenate([Wc, Wd], axis=1), 1, 2)
    xmod = _film_tc(xT, tT, Wab, Wcd)
    m2 = mask.reshape(_B, _L2)
    t2 = t.reshape(_B, _L2)
    xs, ms, ts = _partition_sc(xmod, m2, t2)
    return xs, ms.reshape(_B, _L2, 1), ts.reshape(_B, _L2, 1)


# speedup vs baseline: 17.7234x; 1.0186x over previous
import functools

import numpy as np
import jax
import jax.numpy as jnp
from jax import lax
from jax.experimental import pallas as pl
from jax.experimental.pallas import tpu as pltpu
from jax.experimental.pallas import tpu_sc as plsc

_NUM_BANDS = 2
_EMB = 64
_NH = 4
_TMAX = 1000.0
_B, _L = 16, 2048
_L2 = _L * _NUM_BANDS
_CH = 1024
_HARMONICS = np.arange(1, _NH + 1, dtype=np.float32) * np.float32(
    2.0 * np.pi / _TMAX)


def _film_body(xt_ref, tt_ref, w_ref, o_ref):
    tb = tt_ref[0, 0]
    xb = xt_ref[0, 0]
    w0 = float(_HARMONICS[0])
    s1 = jnp.sin(tb * w0)
    c1 = jnp.cos(tb * w0)
    s2 = 2.0 * s1 * c1
    c2 = 1.0 - 2.0 * s1 * s1
    s3 = s2 * c1 + c2 * s1
    c3 = c2 * c1 - s2 * s1
    s4 = 2.0 * s2 * c2
    c4 = 1.0 - 2.0 * s2 * s2
    sc = jnp.concatenate([s1, s2, s3, s4, c1, c2, c3, c4], axis=0)
    scx = jnp.concatenate([sc * xb, sc], axis=0)
    o_ref[0] = lax.dot_general(scx, w_ref[0], (((0,), (1,)), ((), ())),
                               preferred_element_type=jnp.float32)


def _film_tc(xT, tT, W):
    return pl.pallas_call(
        _film_body,
        grid=(_B, _NUM_BANDS),
        in_specs=[
            pl.BlockSpec((1, 1, 1, _L), lambda b, k: (k, b, 0, 0)),
            pl.BlockSpec((1, 1, 1, _L), lambda b, k: (k, b, 0, 0)),
            pl.BlockSpec((1, _EMB, 4 * _NH), lambda b, k: (k, 0, 0)),
        ],
        out_specs=pl.BlockSpec((1, _L, _EMB), lambda b, k: (b, k, 0)),
        out_shape=jax.ShapeDtypeStruct((_B, _L2, _EMB), jnp.float32),
    )(xT, tT, W)


_DET_UNROLL = 8


def _has_zero(load_row, nrows):
    def det_body(r, acc):
        base = r * _DET_UNROLL
        for u in range(_DET_UNROLL):
            acc = jnp.minimum(acc, jnp.abs(load_row(base + u)))
        return acc
    acc = lax.fori_loop(0, nrows // _DET_UNROLL, det_body,
                        jnp.full((16,), 3.0e38, jnp.float32))
    return jnp.min(acc) == 0.0


def _partition_body(xmod_hbm, m_hbm, t_hbm, xs_hbm, ms_hbm, ts_hbm,
                    inbuf0, inbuf1, outbuf, colin, colout, cnz_ref, cz_ref,
                    sin0, sin1, sout0, sout1):
    cid = lax.axis_index("c")
    sid = lax.axis_index("s")
    wid = sid * 2 + cid
    lanes = lax.iota(jnp.int32, 16)
    bufs = (inbuf0, inbuf1)
    sins = (sin0, sin1)
    souts = (sout0, sout1)
    nch = _L2 // _CH

    for task in range(2):
        tid = wid * 2 + task
        b = tid // 4
        col0 = (tid % 4) * 16

        def src(ci):
            return xmod_hbm.at[b, pl.ds(ci * _CH, _CH), pl.ds(col0, 16)]

        def dst(ci):
            return xs_hbm.at[b, pl.ds(ci * _CH, _CH), pl.ds(col0, 16)]

        clean = jnp.bool_(True)
        fast_flags = []
        in_handles = {0: pltpu.async_copy(src(0), bufs[0], sins[0])}
        out_handles = {}
        for ci in range(nch):
            buf = bufs[ci % 2]
            if ci + 1 < nch:
                if ci >= 1:
                    @pl.when(fast_flags[ci - 1])
                    def _():
                        out_handles[ci - 1].wait()
                in_handles[ci + 1] = pltpu.async_copy(
                    src(ci + 1), bufs[(ci + 1) % 2], sins[(ci + 1) % 2])
            in_handles[ci].wait()
            clean_now = jnp.logical_and(
                clean, jnp.logical_not(_has_zero(lambda r: buf[r], _CH)))

            @pl.when(clean_now)
            def _():
                out_handles[ci] = pltpu.async_copy(
                    buf, dst(ci), souts[ci % 2])

            @pl.when(jnp.logical_not(clean_now))
            def _(clean=clean, ci=ci, buf=buf):
                cnz0 = jnp.where(clean,
                                 jnp.full((16,), ci * _CH, jnp.int32),
                                 cnz_ref[...])
                cz0 = jnp.where(clean, jnp.zeros((16,), jnp.int32),
                                cz_ref[...])

                def row_body(r, carr):
                    cnz, cz = carr
                    v = buf[r]
                    nz = v != 0.0
                    one = jnp.where(nz, 1, 0).astype(jnp.int32)
                    dst_v = jnp.where(nz, cnz, (_L2 - 1) - cz)
                    plsc.store_scatter(outbuf, [dst_v, lanes], v)
                    return (cnz + one, cz + (1 - one))

                cnz1, cz1 = lax.fori_loop(0, _CH, row_body, (cnz0, cz0))
                cnz_ref[...] = cnz1
                cz_ref[...] = cz1

            clean = clean_now
            fast_flags.append(clean_now)
        for ci in (nch - 2, nch - 1):
            @pl.when(fast_flags[ci])
            def _(ci=ci):
                out_handles[ci].wait()
        for ci in range(nch):
            @pl.when(jnp.logical_not(fast_flags[ci]))
            def _(ci=ci):
                pltpu.sync_copy(outbuf.at[pl.ds(ci * _CH, _CH)], dst(ci))

    def column_task(src_hbm, dst_hbm, row):
        pltpu.sync_copy(src_hbm.at[row], colin)
        z = _has_zero(lambda r: colin[pl.ds(r * 16, 16)], _L2 // 16)

        @pl.when(jnp.logical_not(z))
        def _():
            pltpu.sync_copy(colin, dst_hbm.at[row])

        @pl.when(z)
        def _():
            def chunk_body(k, carr):
                cnz, cz = carr
                v = colin[pl.ds(k * 16, 16)]
                nz = v != 0.0
                one = jnp.where(nz, 1, 0).astype(jnp.int32)
                inc = plsc.cumsum(one)
                dst = jnp.where(nz, cnz + inc - 1,
                                _L2 - 1 - cz - lanes + inc)
                plsc.store_scatter(colout, [dst], v)
                tot = jnp.sum(one)
                return (cnz + tot, cz + (16 - tot))

            lax.fori_loop(0, _L2 // 16, chunk_body,
                          (jnp.int32(0), jnp.int32(0)))
            pltpu.sync_copy(colout, dst_hbm.at[row])

    @pl.when(wid < 16)
    def _():
        column_task(m_hbm, ms_hbm, wid)

    @pl.when(wid >= 16)
    def _():
        column_task(t_hbm, ts_hbm, wid - 16)


def _partition_sc(xmod, m2, t2):
    mesh = plsc.VectorSubcoreMesh(core_axis_name="c", subcore_axis_name="s")
    f32 = jnp.float32
    run = functools.partial(
        pl.kernel,
        mesh=mesh,
        compiler_params=pltpu.CompilerParams(
            use_tc_tiling_on_sc=False, needs_layout_passes=False),
        out_type=(
            jax.ShapeDtypeStruct((_B, _L2, _EMB), f32),
            jax.ShapeDtypeStruct((_B, _L2), f32),
            jax.ShapeDtypeStruct((_B, _L2), f32),
        ),
        scratch_types=[
            pltpu.VMEM((_CH, 16), f32),
            pltpu.VMEM((_CH, 16), f32),
            pltpu.VMEM((_L2, 16), f32),
            pltpu.VMEM((_L2,), f32),
            pltpu.VMEM((_L2,), f32),
            pltpu.VMEM((16,), jnp.int32),
            pltpu.VMEM((16,), jnp.int32),
            pltpu.SemaphoreType.DMA,
            pltpu.SemaphoreType.DMA,
            pltpu.SemaphoreType.DMA,
            pltpu.SemaphoreType.DMA,
        ],
    )(_partition_body)
    return run(xmod, m2, t2)


def kernel(x, t, mask, Wa, Wb, Wc, Wd, Wp):
    xT = jnp.transpose(x, (2, 0, 1)).reshape(_NUM_BANDS, _B, 1, _L)
    tT = jnp.transpose(t, (2, 0, 1)).reshape(_NUM_BANDS, _B, 1, _L)
    W = jnp.swapaxes(
        jnp.concatenate([Wa * Wp, Wb * Wp, Wc, Wd], axis=1), 1, 2)
    xmod = _film_tc(xT, tT, W)
    m2 = mask.reshape(_B, _L2)
    t2 = t.reshape(_B, _L2)
    xs, ms, ts = _partition_sc(xmod, m2, t2)
    return xs, ms.reshape(_B, _L2, 1), ts.reshape(_B, _L2, 1)


# speedup vs baseline: 20.6498x; 1.1651x over previous
import functools

import numpy as np
import jax
import jax.numpy as jnp
from jax import lax
from jax.experimental import pallas as pl
from jax.experimental.pallas import tpu as pltpu
from jax.experimental.pallas import tpu_sc as plsc

_NUM_BANDS = 2
_EMB = 64
_NH = 4
_TMAX = 1000.0
_B, _L = 16, 2048
_L2 = _L * _NUM_BANDS
_CH = 128
_HARMONICS = np.arange(1, _NH + 1, dtype=np.float32) * np.float32(
    2.0 * np.pi / _TMAX)


def _film_body(xt_ref, tt_ref, w_ref, o_ref):
    tb = tt_ref[0, 0]
    xb = xt_ref[0, 0]
    w0 = float(_HARMONICS[0])
    s1 = jnp.sin(tb * w0)
    c1 = jnp.cos(tb * w0)
    s2 = 2.0 * s1 * c1
    c2 = 1.0 - 2.0 * s1 * s1
    s3 = s2 * c1 + c2 * s1
    c3 = c2 * c1 - s2 * s1
    s4 = 2.0 * s2 * c2
    c4 = 1.0 - 2.0 * s2 * s2
    sc = jnp.concatenate([s1, s2, s3, s4, c1, c2, c3, c4], axis=0)
    scx = jnp.concatenate([sc * xb, sc], axis=0)
    o_ref[0] = lax.dot_general(scx, w_ref[0], (((0,), (1,)), ((), ())),
                               preferred_element_type=jnp.float32)


def _film_tc(xT, tT, W):
    return pl.pallas_call(
        _film_body,
        grid=(_B, _NUM_BANDS),
        in_specs=[
            pl.BlockSpec((1, 1, 1, _L), lambda b, k: (k, b, 0, 0)),
            pl.BlockSpec((1, 1, 1, _L), lambda b, k: (k, b, 0, 0)),
            pl.BlockSpec((1, _EMB, 4 * _NH), lambda b, k: (k, 0, 0)),
        ],
        out_specs=pl.BlockSpec((1, _L, _EMB), lambda b, k: (b, k, 0)),
        out_shape=jax.ShapeDtypeStruct((_B, _L2, _EMB), jnp.float32),
    )(xT, tT, W)


_DET_UNROLL = 8


def _has_zero(load_row, nrows):
    def det_body(r, acc):
        base = r * _DET_UNROLL
        for u in range(_DET_UNROLL):
            acc = jnp.minimum(acc, jnp.abs(load_row(base + u)))
        return acc
    acc = lax.fori_loop(0, nrows // _DET_UNROLL, det_body,
                        jnp.full((16,), 3.0e38, jnp.float32))
    return jnp.min(acc) == 0.0


def _partition_body(xmod_hbm, m_hbm, t_hbm, xs_hbm, ms_hbm, ts_hbm,
                    inbuf0, inbuf1, outbuf, colin, colout,
                    sin0, sin1, sout0, sout1):
    cid = lax.axis_index("c")
    sid = lax.axis_index("s")
    wid = sid * 2 + cid
    lanes = lax.iota(jnp.int32, 16)
    bufs = (inbuf0, inbuf1)
    sins = (sin0, sin1)
    souts = (sout0, sout1)
    nch = _L2 // _CH

    @pl.when(wid < 16)
    def _():
        b = wid

        def src(ci):
            return xmod_hbm.at[b, pl.ds(ci * _CH, _CH), :]

        def dst(ci):
            return xs_hbm.at[b, pl.ds(ci * _CH, _CH), :]

        def detect(buf):
            def load_row(r):
                rr = r // 4
                g = r % 4
                return buf[rr, pl.ds(g * 16, 16)]
            return _has_zero(load_row, 4 * _CH)

        clean = jnp.bool_(True)
        fast_flags = []
        in_handles = {0: pltpu.async_copy(src(0), bufs[0], sins[0])}
        out_handles = {}
        for ci in range(nch):
            buf = bufs[ci % 2]
            if ci + 1 < nch:
                if ci >= 1:
                    @pl.when(fast_flags[ci - 1])
                    def _():
                        out_handles[ci - 1].wait()
                in_handles[ci + 1] = pltpu.async_copy(
                    src(ci + 1), bufs[(ci + 1) % 2], sins[(ci + 1) % 2])
            in_handles[ci].wait()
            clean_now = jnp.logical_and(
                clean, jnp.logical_not(detect(buf)))

            @pl.when(clean_now)
            def _():
                out_handles[ci] = pltpu.async_copy(
                    buf, dst(ci), souts[ci % 2])

            clean = clean_now
            fast_flags.append(clean_now)
        for ci in (nch - 2, nch - 1):
            @pl.when(fast_flags[ci])
            def _(ci=ci):
                out_handles[ci].wait()

        d = jnp.int32(0)
        for f in fast_flags:
            d = d + jnp.where(f, 1, 0).astype(jnp.int32)

        @pl.when(jnp.logical_not(fast_flags[-1]))
        def _():
            for g in range(4):
                def scatter_chunk(ci, carr):
                    cnz, cz = carr
                    pltpu.sync_copy(src(ci), inbuf0)

                    def row_body(r, carr2):
                        cnz2, cz2 = carr2
                        v = inbuf0[r, pl.ds(g * 16, 16)]
                        nz = v != 0.0
                        one = jnp.where(nz, 1, 0).astype(jnp.int32)
                        dst_v = jnp.where(nz, cnz2, (_L2 - 1) - cz2)
                        p = dst_v * 16 + lanes
                        plsc.store_scatter(outbuf, [p // 128, p % 128], v)
                        return (cnz2 + one, cz2 + (1 - one))

                    return lax.fori_loop(0, _CH, row_body, (cnz, cz))

                cnz0 = jnp.full((16,), d * _CH, jnp.int32)
                cz0 = jnp.zeros((16,), jnp.int32)
                lax.fori_loop(d, nch, scatter_chunk, (cnz0, cz0))

                def merge_chunk(ci, carry):
                    pltpu.sync_copy(dst(ci), inbuf0)

                    def mrow(r, c2):
                        q = ci * _CH + r
                        inbuf0[r, pl.ds(g * 16, 16)] = outbuf[
                            q // 8, pl.ds((q % 8) * 16, 16)]
                        return c2

                    lax.fori_loop(0, _CH, mrow, jnp.int32(0))
                    pltpu.sync_copy(inbuf0, dst(ci))
                    return carry

                lax.fori_loop(d, nch, merge_chunk, jnp.int32(0))

    def column_task(src_hbm, dst_hbm, row):
        pltpu.sync_copy(src_hbm.at[row], colin)
        z = _has_zero(lambda r: colin[pl.ds(r * 16, 16)], _L2 // 16)

        @pl.when(jnp.logical_not(z))
        def _():
            pltpu.sync_copy(colin, dst_hbm.at[row])

        @pl.when(z)
        def _():
            def chunk_body(k, carr):
                cnz, cz = carr
                v = colin[pl.ds(k * 16, 16)]
                nz = v != 0.0
                one = jnp.where(nz, 1, 0).astype(jnp.int32)
                inc = plsc.cumsum(one)
                dst = jnp.where(nz, cnz + inc - 1,
                                _L2 - 1 - cz - lanes + inc)
                plsc.store_scatter(colout, [dst], v)
                tot = jnp.sum(one)
                return (cnz + tot, cz + (16 - tot))

            lax.fori_loop(0, _L2 // 16, chunk_body,
                          (jnp.int32(0), jnp.int32(0)))
            pltpu.sync_copy(colout, dst_hbm.at[row])

    @pl.when(wid >= 16)
    def _():
        column_task(m_hbm, ms_hbm, wid - 16)
        column_task(t_hbm, ts_hbm, wid - 16)


def _partition_sc(xmod, m2, t2):
    mesh = plsc.VectorSubcoreMesh(core_axis_name="c", subcore_axis_name="s")
    f32 = jnp.float32
    run = functools.partial(
        pl.kernel,
        mesh=mesh,
        compiler_params=pltpu.CompilerParams(
            use_tc_tiling_on_sc=True, needs_layout_passes=False),
        out_type=(
            jax.ShapeDtypeStruct((_B, _L2, _EMB), f32),
            jax.ShapeDtypeStruct((_B, _L2), f32),
            jax.ShapeDtypeStruct((_B, _L2), f32),
        ),
        scratch_types=[
            pltpu.VMEM((_CH, _EMB), f32),
            pltpu.VMEM((_CH, _EMB), f32),
            pltpu.VMEM((_L2 * 16 // 128, 128), f32),
            pltpu.VMEM((_L2,), f32),
            pltpu.VMEM((_L2,), f32),
            pltpu.SemaphoreType.DMA,
            pltpu.SemaphoreType.DMA,
            pltpu.SemaphoreType.DMA,
            pltpu.SemaphoreType.DMA,
        ],
    )(_partition_body)
    return run(xmod, m2, t2)


def kernel(x, t, mask, Wa, Wb, Wc, Wd, Wp):
    xT = jnp.transpose(x, (2, 0, 1)).reshape(_NUM_BANDS, _B, 1, _L)
    tT = jnp.transpose(t, (2, 0, 1)).reshape(_NUM_BANDS, _B, 1, _L)
    W = jnp.swapaxes(
        jnp.concatenate([Wa * Wp, Wb * Wp, Wc, Wd], axis=1), 1, 2)
    xmod = _film_tc(xT, tT, W)
    m2 = mask.reshape(_B, _L2)
    t2 = t.reshape(_B, _L2)
    xs, ms, ts = _partition_sc(xmod, m2, t2)
    return xs, ms.reshape(_B, _L2, 1), ts.reshape(_B, _L2, 1)
